# baseline (speedup 1.0000x reference)
import jax
import jax.numpy as jnp
from jax.experimental import pallas as pl

_S = 14.0


def _loss_kernel(p_ref, t_ref, tot_ref, loc_ref, cls_ref):
    p_nat = p_ref[...]
    t_nat = t_ref[...]
    xp = jnp.transpose(p_nat[:, :10, :], (1, 0, 2))
    xt = jnp.transpose(t_nat[:, :10, :], (1, 0, 2))

    def ch(arr, c):
        return arr[c]

    t4 = ch(xt, 4)
    coo = (t4 > 0.0).astype(jnp.float32)
    noo = (t4 == 0.0).astype(jnp.float32)

    d4 = ch(xp, 4) - t4
    d9 = ch(xp, 9) - ch(xt, 9)
    noo_loss = jnp.sum(noo * (d4 * d4 + d9 * d9))

    tx = ch(xt, 0) / _S
    ty = ch(xt, 1) / _S
    tw = ch(xt, 2)
    th = ch(xt, 3)
    t_ltx = tx - 0.5 * tw
    t_lty = ty - 0.5 * th
    t_rbx = tx + 0.5 * tw
    t_rby = ty + 0.5 * th
    area2 = (t_rbx - t_ltx) * (t_rby - t_lty)

    def iou(off):
        px = ch(xp, off) / _S
        py = ch(xp, off + 1) / _S
        pw = ch(xp, off + 2)
        ph = ch(xp, off + 3)
        p_ltx = px - 0.5 * pw
        p_lty = py - 0.5 * ph
        p_rbx = px + 0.5 * pw
        p_rby = py + 0.5 * ph
        ltx = jnp.maximum(p_ltx, t_ltx)
        lty = jnp.maximum(p_lty, t_lty)
        rbx = jnp.minimum(p_rbx, t_rbx)
        rby = jnp.minimum(p_rby, t_rby)
        whx = jnp.maximum(rbx - ltx, 0.0)
        why = jnp.maximum(rby - lty, 0.0)
        inter = whx * why
        area1 = (p_rbx - p_ltx) * (p_rby - p_lty)
        return inter / (area1 + area2 - inter)

    iou0 = iou(0)
    iou1 = iou(5)
    sel = iou1 > iou0
    max_iou = jnp.maximum(iou0, iou1)

    def pick(arr, c):
        return jnp.where(sel, ch(arr, 5 + c), ch(arr, c))

    rp_x = pick(xp, 0)
    rp_y = pick(xp, 1)
    rp_w = pick(xp, 2)
    rp_h = pick(xp, 3)
    rp_c = pick(xp, 4)
    rt_x = pick(xt, 0)
    rt_y = pick(xt, 1)
    rt_w = pick(xt, 2)
    rt_h = pick(xt, 3)
    np_c = jnp.where(sel, ch(xp, 4), ch(xp, 9))

    dx = rp_x - rt_x
    dy = rp_y - rt_y
    dw = jnp.sqrt(rp_w) - jnp.sqrt(rt_w)
    dh = jnp.sqrt(rp_h) - jnp.sqrt(rt_h)
    loc = jnp.sum(coo * (dx * dx + dy * dy + dw * dw + dh * dh))
    dc = rp_c - max_iou
    contain = jnp.sum(coo * dc * dc)
    not_contain = jnp.sum(coo * np_c * np_c)

    cdiff = p_nat[:, 10:30, :] - t_nat[:, 10:30, :]
    cls = jnp.sum(coo[:, None, :] * cdiff * cdiff)

    total = (5.0 * loc + 2.0 * contain + not_contain + 0.5 * noo_loss + cls) * (
        1.0 / 64.0
    )

    @pl.when(pl.program_id(0) == 0)
    def _init():
        tot_ref[...] = jnp.full((1, 1), total)
        loc_ref[...] = jnp.full((1, 1), 5.0 * loc)
        cls_ref[...] = jnp.full((1, 1), cls)

    @pl.when(pl.program_id(0) != 0)
    def _acc():
        tot_ref[...] += jnp.full((1, 1), total)
        loc_ref[...] += jnp.full((1, 1), 5.0 * loc)
        cls_ref[...] += jnp.full((1, 1), cls)


_BLK = 49


def kernel(pred_tensor, target_tensor):
    p = jnp.transpose(pred_tensor, (1, 2, 3, 0)).reshape(196, 30, 64)
    t = jnp.transpose(target_tensor, (1, 2, 3, 0)).reshape(196, 30, 64)
    out_sds = jax.ShapeDtypeStruct((1, 1), jnp.float32)
    in_spec = pl.BlockSpec((_BLK, 30, 64), lambda i: (i, 0, 0))
    out_spec = pl.BlockSpec((1, 1), lambda i: (0, 0))
    tot, loc, cls = pl.pallas_call(
        _loss_kernel,
        grid=(196 // _BLK,),
        in_specs=(in_spec, in_spec),
        out_specs=(out_spec, out_spec, out_spec),
        out_shape=(out_sds, out_sds, out_sds),
    )(p, t)
    return tot[0, 0], loc[0, 0], cls[0, 0]


# speedup vs baseline: 1.2188x; 1.2188x over previous
import jax
import jax.numpy as jnp
from jax.experimental import pallas as pl

_S = 14.0


def _loss_kernel(p_ref, t_ref, tot_ref, loc_ref, cls_ref):
    p_nat = p_ref[...]
    t_nat = t_ref[...]
    xp = jnp.transpose(p_nat[:, :10, :], (1, 0, 2))
    xt = jnp.transpose(t_nat[:, :10, :], (1, 0, 2))

    def ch(arr, c):
        return arr[c]

    t4 = ch(xt, 4)
    coo = (t4 > 0.0).astype(jnp.float32)
    noo = (t4 == 0.0).astype(jnp.float32)

    d4 = ch(xp, 4) - t4
    d9 = ch(xp, 9) - ch(xt, 9)
    noo_loss = jnp.sum(noo * (d4 * d4 + d9 * d9))

    tx = ch(xt, 0) / _S
    ty = ch(xt, 1) / _S
    tw = ch(xt, 2)
    th = ch(xt, 3)
    t_ltx = tx - 0.5 * tw
    t_lty = ty - 0.5 * th
    t_rbx = tx + 0.5 * tw
    t_rby = ty + 0.5 * th
    area2 = (t_rbx - t_ltx) * (t_rby - t_lty)

    def iou(off):
        px = ch(xp, off) / _S
        py = ch(xp, off + 1) / _S
        pw = ch(xp, off + 2)
        ph = ch(xp, off + 3)
        p_ltx = px - 0.5 * pw
        p_lty = py - 0.5 * ph
        p_rbx = px + 0.5 * pw
        p_rby = py + 0.5 * ph
        ltx = jnp.maximum(p_ltx, t_ltx)
        lty = jnp.maximum(p_lty, t_lty)
        rbx = jnp.minimum(p_rbx, t_rbx)
        rby = jnp.minimum(p_rby, t_rby)
        whx = jnp.maximum(rbx - ltx, 0.0)
        why = jnp.maximum(rby - lty, 0.0)
        inter = whx * why
        area1 = (p_rbx - p_ltx) * (p_rby - p_lty)
        return inter / (area1 + area2 - inter)

    iou0 = iou(0)
    iou1 = iou(5)
    sel = iou1 > iou0
    max_iou = jnp.maximum(iou0, iou1)

    def pick(arr, c):
        return jnp.where(sel, ch(arr, 5 + c), ch(arr, c))

    rp_x = pick(xp, 0)
    rp_y = pick(xp, 1)
    rp_w = pick(xp, 2)
    rp_h = pick(xp, 3)
    rp_c = pick(xp, 4)
    rt_x = pick(xt, 0)
    rt_y = pick(xt, 1)
    rt_w = pick(xt, 2)
    rt_h = pick(xt, 3)
    np_c = jnp.where(sel, ch(xp, 4), ch(xp, 9))

    dx = rp_x - rt_x
    dy = rp_y - rt_y
    dw = jnp.sqrt(rp_w) - jnp.sqrt(rt_w)
    dh = jnp.sqrt(rp_h) - jnp.sqrt(rt_h)
    loc = jnp.sum(coo * (dx * dx + dy * dy + dw * dw + dh * dh))
    dc = rp_c - max_iou
    contain = jnp.sum(coo * dc * dc)
    not_contain = jnp.sum(coo * np_c * np_c)

    cdiff = p_nat[:, 10:30, :] - t_nat[:, 10:30, :]
    cls = jnp.sum(coo[:, None, :] * cdiff * cdiff)

    total = (5.0 * loc + 2.0 * contain + not_contain + 0.5 * noo_loss + cls) * (
        1.0 / 64.0
    )

    @pl.when(pl.program_id(0) == 0)
    def _init():
        tot_ref[...] = jnp.full((1, 1), total)
        loc_ref[...] = jnp.full((1, 1), 5.0 * loc)
        cls_ref[...] = jnp.full((1, 1), cls)

    @pl.when(pl.program_id(0) != 0)
    def _acc():
        tot_ref[...] += jnp.full((1, 1), total)
        loc_ref[...] += jnp.full((1, 1), 5.0 * loc)
        cls_ref[...] += jnp.full((1, 1), cls)


_BLK = 98


def kernel(pred_tensor, target_tensor):
    p = jnp.transpose(pred_tensor, (1, 2, 3, 0)).reshape(196, 30, 64)
    t = jnp.transpose(target_tensor, (1, 2, 3, 0)).reshape(196, 30, 64)
    out_sds = jax.ShapeDtypeStruct((1, 1), jnp.float32)
    in_spec = pl.BlockSpec((_BLK, 30, 64), lambda i: (i, 0, 0))
    out_spec = pl.BlockSpec((1, 1), lambda i: (0, 0))
    tot, loc, cls = pl.pallas_call(
        _loss_kernel,
        grid=(196 // _BLK,),
        in_specs=(in_spec, in_spec),
        out_specs=(out_spec, out_spec, out_spec),
        out_shape=(out_sds, out_sds, out_sds),
    )(p, t)
    return tot[0, 0], loc[0, 0], cls[0, 0]


# speedup vs baseline: 1.2380x; 1.0157x over previous
import jax
import jax.numpy as jnp
from jax.experimental import pallas as pl

_S = 14.0


def _loss_kernel(p_ref, t_ref, tot_ref, loc_ref, cls_ref):
    p_nat = p_ref[...]
    t_nat = t_ref[...]
    xp = jnp.transpose(p_nat[:, :10, :], (1, 0, 2))
    xt = jnp.transpose(t_nat[:, :10, :], (1, 0, 2))

    def ch(arr, c):
        return arr[c]

    t4 = ch(xt, 4)
    coo = (t4 > 0.0).astype(jnp.float32)
    noo = (t4 == 0.0).astype(jnp.float32)

    d4 = ch(xp, 4) - t4
    d9 = ch(xp, 9) - ch(xt, 9)
    noo_plane = noo * (d4 * d4 + d9 * d9)

    tx = ch(xt, 0) / _S
    ty = ch(xt, 1) / _S
    tw = ch(xt, 2)
    th = ch(xt, 3)
    t_ltx = tx - 0.5 * tw
    t_lty = ty - 0.5 * th
    t_rbx = tx + 0.5 * tw
    t_rby = ty + 0.5 * th
    area2 = (t_rbx - t_ltx) * (t_rby - t_lty)

    def iou(off):
        px = ch(xp, off) / _S
        py = ch(xp, off + 1) / _S
        pw = ch(xp, off + 2)
        ph = ch(xp, off + 3)
        p_ltx = px - 0.5 * pw
        p_lty = py - 0.5 * ph
        p_rbx = px + 0.5 * pw
        p_rby = py + 0.5 * ph
        ltx = jnp.maximum(p_ltx, t_ltx)
        lty = jnp.maximum(p_lty, t_lty)
        rbx = jnp.minimum(p_rbx, t_rbx)
        rby = jnp.minimum(p_rby, t_rby)
        whx = jnp.maximum(rbx - ltx, 0.0)
        why = jnp.maximum(rby - lty, 0.0)
        inter = whx * why
        area1 = (p_rbx - p_ltx) * (p_rby - p_lty)
        return inter / (area1 + area2 - inter)

    iou0 = iou(0)
    iou1 = iou(5)
    sel = iou1 > iou0
    max_iou = jnp.maximum(iou0, iou1)

    def pick(arr, c):
        return jnp.where(sel, ch(arr, 5 + c), ch(arr, c))

    rp_x = pick(xp, 0)
    rp_y = pick(xp, 1)
    rp_w = pick(xp, 2)
    rp_h = pick(xp, 3)
    rp_c = pick(xp, 4)
    rt_x = pick(xt, 0)
    rt_y = pick(xt, 1)
    rt_w = pick(xt, 2)
    rt_h = pick(xt, 3)
    np_c = jnp.where(sel, ch(xp, 4), ch(xp, 9))

    dx = rp_x - rt_x
    dy = rp_y - rt_y
    dw = jnp.sqrt(rp_w) - jnp.sqrt(rt_w)
    dh = jnp.sqrt(rp_h) - jnp.sqrt(rt_h)
    loc = jnp.sum(coo * (dx * dx + dy * dy + dw * dw + dh * dh))
    dc = rp_c - max_iou
    other = jnp.sum(
        coo * (2.0 * dc * dc + np_c * np_c) + 0.5 * noo_plane
    )

    cdiff = p_nat[:, 10:30, :] - t_nat[:, 10:30, :]
    cls = jnp.sum(coo[:, None, :] * cdiff * cdiff)

    total = (5.0 * loc + other + cls) * (1.0 / 64.0)

    @pl.when(pl.program_id(0) == 0)
    def _init():
        tot_ref[...] = jnp.full((1, 1), total)
        loc_ref[...] = jnp.full((1, 1), 5.0 * loc)
        cls_ref[...] = jnp.full((1, 1), cls)

    @pl.when(pl.program_id(0) != 0)
    def _acc():
        tot_ref[...] += jnp.full((1, 1), total)
        loc_ref[...] += jnp.full((1, 1), 5.0 * loc)
        cls_ref[...] += jnp.full((1, 1), cls)


_BLK = 98


def kernel(pred_tensor, target_tensor):
    p = jnp.transpose(pred_tensor, (1, 2, 3, 0)).reshape(196, 30, 64)
    t = jnp.transpose(target_tensor, (1, 2, 3, 0)).reshape(196, 30, 64)
    out_sds = jax.ShapeDtypeStruct((1, 1), jnp.float32)
    in_spec = pl.BlockSpec((_BLK, 30, 64), lambda i: (i, 0, 0))
    out_spec = pl.BlockSpec((1, 1), lambda i: (0, 0))
    tot, loc, cls = pl.pallas_call(
        _loss_kernel,
        grid=(196 // _BLK,),
        in_specs=(in_spec, in_spec),
        out_specs=(out_spec, out_spec, out_spec),
        out_shape=(out_sds, out_sds, out_sds),
    )(p, t)
    return tot[0, 0], loc[0, 0], cls[0, 0]
